# baseline (speedup 1.0000x reference)
import functools

import jax
import jax.numpy as jnp
from jax import lax
from jax.experimental import pallas as pl
from jax.experimental.pallas import tpu as pltpu
from jax.experimental.pallas import tpu_sc as plsc

NWORDS = 1000000
NTAGS = 64
NUM_WORDS = 16384

NC = 2
NS = 16
NW = NC * NS
B_PER_W = NUM_WORDS // NW
CHUNK = 128
NCHUNK = B_PER_W // CHUNK
LANES = 16
NVEC = NTAGS // LANES


def _sc_partial_sums(words, emb_weight):
    mesh = plsc.VectorSubcoreMesh(core_axis_name="c", subcore_axis_name="s")

    @functools.partial(
        pl.kernel,
        mesh=mesh,
        out_type=jax.ShapeDtypeStruct((NW, NTAGS), jnp.float32),
        compiler_params=pltpu.CompilerParams(use_tc_tiling_on_sc=False),
        scratch_types=[
            pltpu.VMEM((NCHUNK, CHUNK), jnp.int32),
            pltpu.VMEM((B_PER_W, NTAGS), jnp.float32),
            pltpu.VMEM((NTAGS,), jnp.float32),
            pltpu.SemaphoreType.DMA,
        ],
    )
    def k(words_hbm, table_hbm, out_hbm, idx_v, rows_v, partial_v, sem):
        wid = lax.axis_index("s") * NC + lax.axis_index("c")
        base = wid * B_PER_W

        for j in range(NCHUNK):
            pltpu.sync_copy(
                words_hbm.at[pl.ds(base + j * CHUNK, CHUNK)], idx_v.at[j]
            )

        copies = [
            pltpu.async_copy(
                table_hbm.at[idx_v.at[j]],
                rows_v.at[pl.ds(j * CHUNK, CHUNK)],
                sem,
            )
            for j in range(NCHUNK)
        ]
        for c in copies:
            c.wait()

        def body(r, accs):
            return tuple(
                accs[v] + rows_v[r, pl.ds(v * LANES, LANES)]
                for v in range(NVEC)
            )

        accs = lax.fori_loop(
            0, B_PER_W, body,
            tuple(jnp.zeros((LANES,), jnp.float32) for _ in range(NVEC)),
            unroll=8,
        )
        for v in range(NVEC):
            partial_v[pl.ds(v * LANES, LANES)] = accs[v]
        pltpu.sync_copy(partial_v, out_hbm.at[wid])

    return k(words, emb_weight)


def _tc_reduce_kernel(p_ref, b_ref, o_ref):
    o_ref[...] = jnp.sum(p_ref[...], axis=0, keepdims=True) + b_ref[...]


def kernel(words, emb_weight, bias):
    partials = _sc_partial_sums(words.astype(jnp.int32), emb_weight)
    out = pl.pallas_call(
        _tc_reduce_kernel,
        out_shape=jax.ShapeDtypeStruct((1, NTAGS), jnp.float32),
    )(partials, bias.reshape(1, NTAGS))
    return out


# speedup vs baseline: 3.3830x; 3.3830x over previous
import functools

import jax
import jax.numpy as jnp
from jax import lax
from jax.experimental import pallas as pl
from jax.experimental.pallas import tpu as pltpu
from jax.experimental.pallas import tpu_sc as plsc

NWORDS = 1000000
NTAGS = 64
NUM_WORDS = 16384

NC = 2
NS = 16
NW = NC * NS
B_PER_W = NUM_WORDS // NW
LANES = 16

CPAD = 1 << 20
SHARD = CPAD // NS
ZCHUNK = 8192
IDX_MINOR = 128
NIDX = B_PER_W // IDX_MINOR

BK = 8192
NSTEP = (NWORDS + BK - 1) // BK


def _sc_counts(words):
    mesh = plsc.VectorSubcoreMesh(core_axis_name="c", subcore_axis_name="s")

    @functools.partial(
        pl.kernel,
        mesh=mesh,
        out_type=jax.ShapeDtypeStruct((NC, CPAD), jnp.float32),
        scratch_types=[
            pltpu.VMEM((ZCHUNK,), jnp.float32),
            pltpu.VMEM((NIDX, IDX_MINOR), jnp.int32),
            pltpu.VMEM((IDX_MINOR,), jnp.float32),
            pltpu.VMEM_SHARED((CPAD,), jnp.float32),
        ],
    )
    def k(words_hbm, out_hbm, zeros_v, idx_v, ones_v, counts_sh):
        cid = lax.axis_index("c")
        sid = lax.axis_index("s")
        wid = sid * NC + cid
        base = wid * B_PER_W

        for j in range(NIDX):
            pltpu.sync_copy(
                words_hbm.at[pl.ds(base + j * IDX_MINOR, IDX_MINOR)],
                idx_v.at[j],
            )
        for t in range(ZCHUNK // LANES):
            zeros_v[pl.ds(t * LANES, LANES)] = jnp.zeros((LANES,), jnp.float32)
        for t in range(IDX_MINOR // LANES):
            ones_v[pl.ds(t * LANES, LANES)] = jnp.ones((LANES,), jnp.float32)

        for t in range(SHARD // ZCHUNK):
            pltpu.sync_copy(
                zeros_v, counts_sh.at[pl.ds(sid * SHARD + t * ZCHUNK, ZCHUNK)]
            )
        plsc.subcore_barrier()

        for j in range(NIDX):
            pltpu.sync_copy(ones_v, counts_sh.at[idx_v.at[j]], add=True)
        plsc.subcore_barrier()

        pltpu.sync_copy(
            counts_sh.at[pl.ds(sid * SHARD, SHARD)],
            out_hbm.at[cid, pl.ds(sid * SHARD, SHARD)],
        )

    return k(words)


def _tc_matvec_kernel(xt_ref, cnt_ref, b_ref, o_ref, acc_ref):
    step = pl.program_id(0)

    @pl.when(step == 0)
    def _init():
        acc_ref[...] = jnp.zeros_like(acc_ref)

    cnt = (cnt_ref[0, :] + cnt_ref[1, :])[None, :]
    prod = jnp.where(cnt != 0.0, xt_ref[...] * cnt, 0.0)
    acc_ref[...] += jnp.sum(
        prod.reshape(NTAGS, BK // 128, 128), axis=1
    )

    @pl.when(step == NSTEP - 1)
    def _done():
        o_ref[...] = jnp.sum(acc_ref[...], axis=1)[None, :] + b_ref[...]


def kernel(words, emb_weight, bias):
    counts = _sc_counts(words.astype(jnp.int32))
    xt = emb_weight.T
    out = pl.pallas_call(
        _tc_matvec_kernel,
        grid=(NSTEP,),
        in_specs=[
            pl.BlockSpec((NTAGS, BK), lambda k: (0, k)),
            pl.BlockSpec((NC, BK), lambda k: (0, k)),
            pl.BlockSpec((1, NTAGS), lambda k: (0, 0)),
        ],
        out_specs=pl.BlockSpec((1, NTAGS), lambda k: (0, 0)),
        scratch_shapes=[pltpu.VMEM((NTAGS, 128), jnp.float32)],
        out_shape=jax.ShapeDtypeStruct((1, NTAGS), jnp.float32),
    )(xt, counts, bias.reshape(1, NTAGS))
    return out


# speedup vs baseline: 4.0232x; 1.1892x over previous
import functools

import jax
import jax.numpy as jnp
from jax import lax
from jax.experimental import pallas as pl
from jax.experimental.pallas import tpu as pltpu
from jax.experimental.pallas import tpu_sc as plsc

NWORDS = 1000000
NTAGS = 64
NUM_WORDS = 16384

NC = 2
NS = 16
NW = NC * NS
B_PER_W = NUM_WORDS // NW
LANES = 16

CPAD = 1 << 20
SHARD = CPAD // NS
ZCHUNK = 8192
IDX_MINOR = 128
NIDX = B_PER_W // IDX_MINOR

BK = 4096
KTAIL = 999424
SPB = 114
CW = 128
K0 = KTAIL - NW * CW * SPB
NSTEP_MAIN = K0 // BK
NGRP = CW // LANES


def _sc_counts(words):
    mesh = plsc.VectorSubcoreMesh(core_axis_name="c", subcore_axis_name="s")

    @functools.partial(
        pl.kernel,
        mesh=mesh,
        out_type=jax.ShapeDtypeStruct((NC, CPAD), jnp.float32),
        scratch_types=[
            pltpu.VMEM((ZCHUNK,), jnp.float32),
            pltpu.VMEM((NIDX, IDX_MINOR), jnp.int32),
            pltpu.VMEM((IDX_MINOR,), jnp.float32),
            pltpu.VMEM_SHARED((CPAD,), jnp.float32),
        ],
    )
    def k(words_hbm, out_hbm, zeros_v, idx_v, ones_v, counts_sh):
        cid = lax.axis_index("c")
        sid = lax.axis_index("s")
        wid = sid * NC + cid
        base = wid * B_PER_W

        for j in range(NIDX):
            pltpu.sync_copy(
                words_hbm.at[pl.ds(base + j * IDX_MINOR, IDX_MINOR)],
                idx_v.at[j],
            )
        for t in range(ZCHUNK // LANES):
            zeros_v[pl.ds(t * LANES, LANES)] = jnp.zeros((LANES,), jnp.float32)
        for t in range(IDX_MINOR // LANES):
            ones_v[pl.ds(t * LANES, LANES)] = jnp.ones((LANES,), jnp.float32)

        for t in range(SHARD // ZCHUNK):
            pltpu.sync_copy(
                zeros_v, counts_sh.at[pl.ds(sid * SHARD + t * ZCHUNK, ZCHUNK)]
            )
        plsc.subcore_barrier()

        for j in range(NIDX):
            pltpu.sync_copy(ones_v, counts_sh.at[idx_v.at[j]], add=True)
        plsc.subcore_barrier()

        pltpu.sync_copy(
            counts_sh.at[pl.ds(sid * SHARD, SHARD)],
            out_hbm.at[cid, pl.ds(sid * SHARD, SHARD)],
        )

    return k(words)


def _sc_matvec(xt, counts):
    mesh = plsc.VectorSubcoreMesh(core_axis_name="c", subcore_axis_name="s")

    @functools.partial(
        pl.kernel,
        mesh=mesh,
        out_type=jax.ShapeDtypeStruct((NW, NTAGS, LANES), jnp.float32),
        scratch_types=[
            pltpu.VMEM((NTAGS, CW), jnp.float32),
            pltpu.VMEM((NTAGS, CW), jnp.float32),
            pltpu.VMEM((NC, CW), jnp.float32),
            pltpu.VMEM((NC, CW), jnp.float32),
            pltpu.VMEM((NTAGS, LANES), jnp.float32),
            pltpu.SemaphoreType.DMA,
            pltpu.SemaphoreType.DMA,
        ],
    )
    def k(xt_hbm, cnt_hbm, out_hbm, xb0, xb1, cb0, cb1, acc_v, sem0, sem1):
        wid = lax.axis_index("s") * NC + lax.axis_index("c")
        wbase = K0 + wid * (CW * SPB)
        xbs = (xb0, xb1)
        cbs = (cb0, cb1)
        sems = (sem0, sem1)

        for t in range(NTAGS):
            acc_v[t, pl.ds(0, LANES)] = jnp.zeros((LANES,), jnp.float32)

        def fire(c, par):
            col = pl.multiple_of(wbase + c * CW, CW)
            pltpu.async_copy(xt_hbm.at[:, pl.ds(col, CW)], xbs[par], sems[par])
            pltpu.async_copy(cnt_hbm.at[:, pl.ds(col, CW)], cbs[par], sems[par])

        def drain(par):
            pltpu.make_async_copy(
                xt_hbm.at[:, pl.ds(0, CW)], xbs[par], sems[par]
            ).wait()
            pltpu.make_async_copy(
                cnt_hbm.at[:, pl.ds(0, CW)], cbs[par], sems[par]
            ).wait()

        def compute(par):
            xb, cb = xbs[par], cbs[par]
            cs = tuple(
                cb[0, pl.ds(g * LANES, LANES)] + cb[1, pl.ds(g * LANES, LANES)]
                for g in range(NGRP)
            )

            def tag(t, carry):
                a = acc_v[t, pl.ds(0, LANES)]
                for g in range(NGRP):
                    a = a + xb[t, pl.ds(g * LANES, LANES)] * cs[g]
                acc_v[t, pl.ds(0, LANES)] = a
                return carry

            lax.fori_loop(0, NTAGS, tag, 0)

        fire(0, 0)
        fire(1, 1)

        def pair(cc, carry):
            c = cc * 2
            drain(0)
            compute(0)
            fire(c + 2, 0)
            drain(1)
            compute(1)
            fire(c + 3, 1)
            return carry

        lax.fori_loop(0, SPB // 2 - 1, pair, 0)
        drain(0)
        compute(0)
        drain(1)
        compute(1)

        pltpu.sync_copy(acc_v, out_hbm.at[wid])

    return k(xt, counts)


def _tc_matvec_kernel(xt_ref, cnt_ref, b_ref, o_ref, acc_ref):
    step = pl.program_id(0)

    @pl.when(step == 0)
    def _init():
        acc_ref[...] = jnp.zeros_like(acc_ref)

    cnt = (cnt_ref[0, :] + cnt_ref[1, :])[None, :]
    prod = jnp.where(cnt != 0.0, xt_ref[...] * cnt, 0.0)
    acc_ref[...] += jnp.sum(
        prod.reshape(NTAGS, BK // 128, 128), axis=1
    )

    @pl.when(step == NSTEP_MAIN)
    def _done():
        o_ref[...] = jnp.sum(acc_ref[...], axis=1)[None, :] + b_ref[...]


def _tc_combine_kernel(t_ref, s_ref, o_ref):
    o_ref[...] = t_ref[...] + jnp.sum(s_ref[...], axis=(0, 2))[None, :]


def kernel(words, emb_weight, bias):
    counts = _sc_counts(words.astype(jnp.int32))
    xt = emb_weight.T
    sc_part = _sc_matvec(xt, counts)
    tc_part = pl.pallas_call(
        _tc_matvec_kernel,
        grid=(NSTEP_MAIN + 1,),
        in_specs=[
            pl.BlockSpec(
                (NTAGS, BK),
                lambda k: (0, jnp.where(k == NSTEP_MAIN, KTAIL // BK, k)),
            ),
            pl.BlockSpec(
                (NC, BK),
                lambda k: (0, jnp.where(k == NSTEP_MAIN, KTAIL // BK, k)),
            ),
            pl.BlockSpec((1, NTAGS), lambda k: (0, 0)),
        ],
        out_specs=pl.BlockSpec((1, NTAGS), lambda k: (0, 0)),
        scratch_shapes=[pltpu.VMEM((NTAGS, 128), jnp.float32)],
        out_shape=jax.ShapeDtypeStruct((1, NTAGS), jnp.float32),
    )(xt, counts, bias.reshape(1, NTAGS))
    out = pl.pallas_call(
        _tc_combine_kernel,
        out_shape=jax.ShapeDtypeStruct((1, NTAGS), jnp.float32),
    )(tc_part, sc_part)
    return out


# speedup vs baseline: 4.3038x; 1.0697x over previous
import functools

import jax
import jax.numpy as jnp
from jax import lax
from jax.experimental import pallas as pl
from jax.experimental.pallas import tpu as pltpu
from jax.experimental.pallas import tpu_sc as plsc

NWORDS = 1000000
NTAGS = 64
NUM_WORDS = 16384

NC = 2
NS = 16
NW = NC * NS
B_PER_W = NUM_WORDS // NW
LANES = 16

CPAD = 1 << 20
SHARD = CPAD // NS
ZCHUNK = 8192
IDX_MINOR = 128
NIDX = B_PER_W // IDX_MINOR

BK = 16384
KTAIL = 999424
SPB = 112
CW = 128
K0 = KTAIL - NW * CW * SPB
NSTEP_MAIN = K0 // BK
NGRP = CW // LANES
NBUF = 4


def _sc_counts(words):
    mesh = plsc.VectorSubcoreMesh(core_axis_name="c", subcore_axis_name="s")

    @functools.partial(
        pl.kernel,
        mesh=mesh,
        out_type=jax.ShapeDtypeStruct((NC, CPAD), jnp.float32),
        scratch_types=[
            pltpu.VMEM((ZCHUNK,), jnp.float32),
            pltpu.VMEM((NIDX, IDX_MINOR), jnp.int32),
            pltpu.VMEM((IDX_MINOR,), jnp.float32),
            pltpu.VMEM_SHARED((CPAD,), jnp.float32),
        ],
    )
    def k(words_hbm, out_hbm, zeros_v, idx_v, ones_v, counts_sh):
        cid = lax.axis_index("c")
        sid = lax.axis_index("s")
        wid = sid * NC + cid
        base = wid * B_PER_W

        for j in range(NIDX):
            pltpu.sync_copy(
                words_hbm.at[pl.ds(base + j * IDX_MINOR, IDX_MINOR)],
                idx_v.at[j],
            )
        for t in range(ZCHUNK // LANES):
            zeros_v[pl.ds(t * LANES, LANES)] = jnp.zeros((LANES,), jnp.float32)
        for t in range(IDX_MINOR // LANES):
            ones_v[pl.ds(t * LANES, LANES)] = jnp.ones((LANES,), jnp.float32)

        for t in range(SHARD // ZCHUNK):
            pltpu.sync_copy(
                zeros_v, counts_sh.at[pl.ds(sid * SHARD + t * ZCHUNK, ZCHUNK)]
            )
        plsc.subcore_barrier()

        for j in range(NIDX):
            pltpu.sync_copy(ones_v, counts_sh.at[idx_v.at[j]], add=True)
        plsc.subcore_barrier()

        pltpu.sync_copy(
            counts_sh.at[pl.ds(sid * SHARD, SHARD)],
            out_hbm.at[cid, pl.ds(sid * SHARD, SHARD)],
        )

    return k(words)


def _sc_matvec(xt, counts):
    mesh = plsc.VectorSubcoreMesh(core_axis_name="c", subcore_axis_name="s")

    @functools.partial(
        pl.kernel,
        mesh=mesh,
        out_type=jax.ShapeDtypeStruct((NW, NTAGS, LANES), jnp.float32),
        scratch_types=(
            [pltpu.VMEM((NTAGS, CW), jnp.float32)] * NBUF
            + [pltpu.VMEM((NC, CW), jnp.float32)] * NBUF
            + [pltpu.VMEM((NTAGS, LANES), jnp.float32)]
            + [pltpu.SemaphoreType.DMA] * NBUF
        ),
    )
    def k(xt_hbm, cnt_hbm, out_hbm, *scr):
        xbs = scr[0:NBUF]
        cbs = scr[NBUF:2 * NBUF]
        acc_v = scr[2 * NBUF]
        sems = scr[2 * NBUF + 1:]
        wid = lax.axis_index("s") * NC + lax.axis_index("c")
        wbase = K0 + wid * (CW * SPB)

        for t in range(NTAGS):
            acc_v[t, pl.ds(0, LANES)] = jnp.zeros((LANES,), jnp.float32)

        def fire(c, par):
            col = pl.multiple_of(wbase + c * CW, CW)
            pltpu.async_copy(xt_hbm.at[:, pl.ds(col, CW)], xbs[par], sems[par])
            pltpu.async_copy(cnt_hbm.at[:, pl.ds(col, CW)], cbs[par], sems[par])

        def drain(par):
            pltpu.make_async_copy(
                xt_hbm.at[:, pl.ds(0, CW)], xbs[par], sems[par]
            ).wait()
            pltpu.make_async_copy(
                cnt_hbm.at[:, pl.ds(0, CW)], cbs[par], sems[par]
            ).wait()

        def compute(par):
            xb, cb = xbs[par], cbs[par]
            cs = tuple(
                cb[0, pl.ds(g * LANES, LANES)] + cb[1, pl.ds(g * LANES, LANES)]
                for g in range(NGRP)
            )

            def tag(t, carry):
                a = acc_v[t, pl.ds(0, LANES)]
                for g in range(NGRP):
                    a = a + xb[t, pl.ds(g * LANES, LANES)] * cs[g]
                acc_v[t, pl.ds(0, LANES)] = a
                return carry

            lax.fori_loop(0, NTAGS, tag, 0, unroll=4)

        for par in range(NBUF):
            fire(par, par)

        def ring(cc, carry):
            c = cc * NBUF
            for par in range(NBUF):
                drain(par)
                compute(par)
                fire(c + par + NBUF, par)
            return carry

        lax.fori_loop(0, SPB // NBUF - 1, ring, 0)
        for par in range(NBUF):
            drain(par)
            compute(par)

        pltpu.sync_copy(acc_v, out_hbm.at[wid])

    return k(xt, counts)


def _tc_matvec_kernel(xt_ref, cnt_ref, b_ref, o_ref, acc_ref):
    step = pl.program_id(0)

    @pl.when(step == 0)
    def _init():
        acc_ref[...] = jnp.zeros_like(acc_ref)

    cnt = (cnt_ref[0, :] + cnt_ref[1, :])[None, :]
    prod = jnp.where(cnt != 0.0, xt_ref[...] * cnt, 0.0)
    acc_ref[...] += jnp.sum(
        prod.reshape(NTAGS, BK // 128, 128), axis=1
    )

    @pl.when(step == NSTEP_MAIN)
    def _done():
        o_ref[...] = jnp.sum(acc_ref[...], axis=1)[None, :] + b_ref[...]


def _tc_combine_kernel(t_ref, s_ref, o_ref):
    o_ref[...] = t_ref[...] + jnp.sum(s_ref[...], axis=(0, 2))[None, :]


def kernel(words, emb_weight, bias):
    counts = _sc_counts(words.astype(jnp.int32))
    xt = emb_weight.T
    sc_part = _sc_matvec(xt, counts)
    tc_part = pl.pallas_call(
        _tc_matvec_kernel,
        grid=(NSTEP_MAIN + 1,),
        in_specs=[
            pl.BlockSpec(
                (NTAGS, BK),
                lambda k: (0, jnp.where(k == NSTEP_MAIN, KTAIL // BK, k)),
            ),
            pl.BlockSpec(
                (NC, BK),
                lambda k: (0, jnp.where(k == NSTEP_MAIN, KTAIL // BK, k)),
            ),
            pl.BlockSpec((1, NTAGS), lambda k: (0, 0)),
        ],
        out_specs=pl.BlockSpec((1, NTAGS), lambda k: (0, 0)),
        scratch_shapes=[pltpu.VMEM((NTAGS, 128), jnp.float32)],
        out_shape=jax.ShapeDtypeStruct((1, NTAGS), jnp.float32),
    )(xt, counts, bias.reshape(1, NTAGS))
    out = pl.pallas_call(
        _tc_combine_kernel,
        out_shape=jax.ShapeDtypeStruct((1, NTAGS), jnp.float32),
    )(tc_part, sc_part)
    return out


# speedup vs baseline: 4.4262x; 1.0284x over previous
import functools

import jax
import jax.numpy as jnp
from jax import lax
from jax.experimental import pallas as pl
from jax.experimental.pallas import tpu as pltpu
from jax.experimental.pallas import tpu_sc as plsc

NWORDS = 1000000
NTAGS = 64
NUM_WORDS = 16384

NC = 2
NS = 16
NW = NC * NS
B_PER_W = NUM_WORDS // NW
LANES = 16

CPAD = 1 << 20
SHARD = CPAD // NS
ZCHUNK = 8192
IDX_MINOR = 128
NIDX = B_PER_W // IDX_MINOR

BK = 16384
KTAIL = 999424
SPB = 48
CW = 256
K0 = KTAIL - NW * CW * SPB
NSTEP_MAIN = K0 // BK
NGRP = CW // LANES
NBUF = 4


def _sc_counts(words):
    mesh = plsc.VectorSubcoreMesh(core_axis_name="c", subcore_axis_name="s")

    @functools.partial(
        pl.kernel,
        mesh=mesh,
        out_type=jax.ShapeDtypeStruct((NC, CPAD), jnp.float32),
        scratch_types=[
            pltpu.VMEM((ZCHUNK,), jnp.float32),
            pltpu.VMEM((NIDX, IDX_MINOR), jnp.int32),
            pltpu.VMEM((IDX_MINOR,), jnp.float32),
            pltpu.VMEM_SHARED((CPAD,), jnp.float32),
        ],
    )
    def k(words_hbm, out_hbm, zeros_v, idx_v, ones_v, counts_sh):
        cid = lax.axis_index("c")
        sid = lax.axis_index("s")
        wid = sid * NC + cid
        base = wid * B_PER_W

        for j in range(NIDX):
            pltpu.sync_copy(
                words_hbm.at[pl.ds(base + j * IDX_MINOR, IDX_MINOR)],
                idx_v.at[j],
            )
        for t in range(ZCHUNK // LANES):
            zeros_v[pl.ds(t * LANES, LANES)] = jnp.zeros((LANES,), jnp.float32)
        for t in range(IDX_MINOR // LANES):
            ones_v[pl.ds(t * LANES, LANES)] = jnp.ones((LANES,), jnp.float32)

        for t in range(SHARD // ZCHUNK):
            pltpu.sync_copy(
                zeros_v, counts_sh.at[pl.ds(sid * SHARD + t * ZCHUNK, ZCHUNK)]
            )
        plsc.subcore_barrier()

        for j in range(NIDX):
            pltpu.sync_copy(ones_v, counts_sh.at[idx_v.at[j]], add=True)
        plsc.subcore_barrier()

        pltpu.sync_copy(
            counts_sh.at[pl.ds(sid * SHARD, SHARD)],
            out_hbm.at[cid, pl.ds(sid * SHARD, SHARD)],
        )

    return k(words)


def _sc_matvec(xt, counts):
    mesh = plsc.VectorSubcoreMesh(core_axis_name="c", subcore_axis_name="s")

    @functools.partial(
        pl.kernel,
        mesh=mesh,
        out_type=jax.ShapeDtypeStruct((NW, NTAGS, LANES), jnp.float32),
        scratch_types=(
            [pltpu.VMEM((NTAGS, CW), jnp.float32)] * NBUF
            + [pltpu.VMEM((NC, CW), jnp.float32)] * NBUF
            + [pltpu.VMEM((NTAGS, LANES), jnp.float32)]
            + [pltpu.SemaphoreType.DMA] * NBUF
        ),
    )
    def k(xt_hbm, cnt_hbm, out_hbm, *scr):
        xbs = scr[0:NBUF]
        cbs = scr[NBUF:2 * NBUF]
        acc_v = scr[2 * NBUF]
        sems = scr[2 * NBUF + 1:]
        wid = lax.axis_index("s") * NC + lax.axis_index("c")
        wbase = K0 + wid * (CW * SPB)

        for t in range(NTAGS):
            acc_v[t, pl.ds(0, LANES)] = jnp.zeros((LANES,), jnp.float32)

        def fire(c, par):
            col = pl.multiple_of(wbase + c * CW, CW)
            pltpu.async_copy(xt_hbm.at[:, pl.ds(col, CW)], xbs[par], sems[par])
            pltpu.async_copy(cnt_hbm.at[:, pl.ds(col, CW)], cbs[par], sems[par])

        def drain(par):
            pltpu.make_async_copy(
                xt_hbm.at[:, pl.ds(0, CW)], xbs[par], sems[par]
            ).wait()
            pltpu.make_async_copy(
                cnt_hbm.at[:, pl.ds(0, CW)], cbs[par], sems[par]
            ).wait()

        def compute(par):
            xb, cb = xbs[par], cbs[par]
            cs = tuple(
                cb[0, pl.ds(g * LANES, LANES)] + cb[1, pl.ds(g * LANES, LANES)]
                for g in range(NGRP)
            )

            def tag(t, carry):
                a = acc_v[t, pl.ds(0, LANES)]
                for g in range(NGRP):
                    a = a + xb[t, pl.ds(g * LANES, LANES)] * cs[g]
                acc_v[t, pl.ds(0, LANES)] = a
                return carry

            lax.fori_loop(0, NTAGS, tag, 0, unroll=8)

        for par in range(NBUF):
            fire(par, par)

        def ring(cc, carry):
            c = cc * NBUF
            for par in range(NBUF):
                drain(par)
                compute(par)
                fire(c + par + NBUF, par)
            return carry

        lax.fori_loop(0, SPB // NBUF - 1, ring, 0)
        for par in range(NBUF):
            drain(par)
            compute(par)

        pltpu.sync_copy(acc_v, out_hbm.at[wid])

    return k(xt, counts)


def _tc_matvec_kernel(xt_ref, cnt_ref, b_ref, o_ref, acc_ref):
    step = pl.program_id(0)

    @pl.when(step == 0)
    def _init():
        acc_ref[...] = jnp.zeros_like(acc_ref)

    cnt = (cnt_ref[0, :] + cnt_ref[1, :])[None, :]
    prod = jnp.where(cnt != 0.0, xt_ref[...] * cnt, 0.0)
    acc_ref[...] += jnp.sum(
        prod.reshape(NTAGS, BK // 128, 128), axis=1
    )

    @pl.when(step == NSTEP_MAIN)
    def _done():
        o_ref[...] = jnp.sum(acc_ref[...], axis=1)[None, :] + b_ref[...]


def _tc_combine_kernel(t_ref, s_ref, o_ref):
    o_ref[...] = t_ref[...] + jnp.sum(s_ref[...], axis=(0, 2))[None, :]


def kernel(words, emb_weight, bias):
    counts = _sc_counts(words.astype(jnp.int32))
    xt = emb_weight.T
    sc_part = _sc_matvec(xt, counts)
    tc_part = pl.pallas_call(
        _tc_matvec_kernel,
        grid=(NSTEP_MAIN + 1,),
        in_specs=[
            pl.BlockSpec(
                (NTAGS, BK),
                lambda k: (0, jnp.where(k == NSTEP_MAIN, KTAIL // BK, k)),
            ),
            pl.BlockSpec(
                (NC, BK),
                lambda k: (0, jnp.where(k == NSTEP_MAIN, KTAIL // BK, k)),
            ),
            pl.BlockSpec((1, NTAGS), lambda k: (0, 0)),
        ],
        out_specs=pl.BlockSpec((1, NTAGS), lambda k: (0, 0)),
        scratch_shapes=[pltpu.VMEM((NTAGS, 128), jnp.float32)],
        out_shape=jax.ShapeDtypeStruct((1, NTAGS), jnp.float32),
    )(xt, counts, bias.reshape(1, NTAGS))
    out = pl.pallas_call(
        _tc_combine_kernel,
        out_shape=jax.ShapeDtypeStruct((1, NTAGS), jnp.float32),
    )(tc_part, sc_part)
    return out
